# Initial kernel scaffold; baseline (speedup 1.0000x reference)
import jax
import jax.numpy as jnp
from jax.experimental import pallas as pl


def kernel(teacher_feats, student_feats):
    raise NotImplementedError("write your pallas kernel here")


# speedup vs baseline: 4.7865x; 4.7865x over previous
import functools

import numpy as np
import jax
import jax.numpy as jnp
from jax import lax
from jax.experimental import pallas as pl
from jax.experimental.pallas import tpu as pltpu
from jax.experimental.pallas import tpu_sc as plsc

B, P, D = 2, 1024, 384
EF = 4
R = 128
S = 64
K = 4
EPS = 1e-8
ROWS_PER_WORKER = (B * K * R) // 32

_REF_PERM = np.array([3, 314, 747, 931, 71, 460, 167, 179, 499, 286, 843, 492, 213, 718, 787, 165, 977, 686, 441, 59, 991, 530, 744, 695, 704, 374, 829, 668, 633, 433, 932, 468, 219, 707, 0, 505, 990, 440, 41, 378, 20, 367, 247, 756, 258, 934, 602, 811, 147, 411, 461, 743, 267, 285, 997, 597, 101, 366, 791, 671, 784, 562, 279, 926, 937, 347, 981, 615, 793, 540, 1016, 185, 302, 445, 953, 1022, 312, 482, 595, 266, 731, 241, 790, 502, 453, 372, 471, 1007, 399, 551, 703, 520, 497, 728, 31, 465, 737, 473, 287, 620, 769, 776, 817, 387, 524, 331, 470, 848, 365, 839, 75, 909, 398, 857, 305, 424, 320, 1020, 292, 755, 992, 946, 952, 294, 212, 6, 939, 541], dtype=np.int32)
_SHARED_PERM = np.array([382, 452, 484, 472, 151, 773, 304, 600, 995, 278, 86, 305, 848, 836, 987, 620, 807, 637, 34, 692, 363, 486, 421, 404, 212, 794, 260, 191, 124, 128, 197, 61, 169, 546, 541, 811, 897, 855, 365, 744, 119, 104, 764, 338, 577, 832, 618, 117, 18, 430, 297, 160, 697, 172, 389, 91, 367, 914, 89, 1014, 750, 249, 560, 294], dtype=np.int32)


def _norm_rows(x):
    n = jnp.sqrt(jnp.sum(x * x, axis=-1, keepdims=True))
    return x / jnp.maximum(n, EPS)


def _topk_body(t_ref, x_ref, idx_ref, sim_sc):
    b = pl.program_id(0)
    f = pl.program_id(1)
    E = t_ref[0, 0]
    X = x_ref[0]
    simf = lax.dot_general(_norm_rows(X), _norm_rows(E),
                           (((1,), (1,)), ((), ())),
                           preferred_element_type=jnp.float32)
    sim_sc[:, pl.ds(f * P, P)] = simf

    @pl.when(f == EF - 1)
    def _():
        s = sim_sc[...]
        col = lax.broadcasted_iota(jnp.int32, (R, EF * P), 1)
        g_rows = []
        for _k in range(K):
            m = jnp.max(s, axis=1, keepdims=True)
            cand = jnp.where(s == m, col, EF * P)
            ik = jnp.min(cand, axis=1)
            s = jnp.where(col == ik[:, None], -jnp.inf, s)
            g = b * (8 * P) + P + ((ik >> 10) << 11) + (ik & (P - 1))
            g_rows.append(g[None, :])
        pad = jnp.zeros((8 - K, R), jnp.int32)
        idx_ref[0] = jnp.concatenate(g_rows + [pad], axis=0)


def _topk_call(teacher, ref_t):
    return pl.pallas_call(
        _topk_body,
        grid=(B, EF),
        in_specs=[
            pl.BlockSpec((1, 1, P, D), lambda b, f: (b, 2 * f + 1, 0, 0)),
            pl.BlockSpec((1, R, D), lambda b, f: (b, 0, 0)),
        ],
        out_specs=pl.BlockSpec((1, 8, R), lambda b, f: (b, 0, 0)),
        out_shape=jax.ShapeDtypeStruct((B, 8, R), jnp.int32),
        scratch_shapes=[pltpu.VMEM((R, EF * P), jnp.float32)],
    )(teacher, ref_t)


def _gather_body(table_hbm, idx_hbm, out_hbm, idx_v, rows_v, sem):
    wid = lax.axis_index("s") * 2 + lax.axis_index("c")
    base = wid * ROWS_PER_WORKER
    pltpu.sync_copy(idx_hbm.at[pl.ds(base, ROWS_PER_WORKER)], idx_v)
    pltpu.async_copy(table_hbm.at[idx_v], rows_v, sem).wait()
    pltpu.sync_copy(rows_v, out_hbm.at[pl.ds(base, ROWS_PER_WORKER)])


def _gather_call(table, idxg):
    mesh = plsc.VectorSubcoreMesh(core_axis_name="c", subcore_axis_name="s")
    k = functools.partial(
        pl.kernel,
        out_type=jax.ShapeDtypeStruct((B * K * R, D), jnp.float32),
        mesh=mesh,
        scratch_types=[
            pltpu.VMEM((ROWS_PER_WORKER,), jnp.int32),
            pltpu.VMEM((ROWS_PER_WORKER, D), jnp.float32),
            pltpu.SemaphoreType.DMA,
        ],
    )(_gather_body)
    return k(table, idxg)


def _loss_body(z_ref, xt_ref, xs_ref, sht_ref, shs_ref, out_ref):
    Zall = z_ref[0]
    xt = xt_ref[0]
    xs = xs_ref[0]
    XXt = jnp.sum(xt * xt, axis=1)
    XXs = jnp.sum(xs * xs, axis=1)
    acc = jnp.float32(0.0)
    for f in range(3):
        Yt = sht_ref[0, f]
        Ys = shs_ref[0, f]
        YYt = jnp.sum(Yt * Yt, axis=1)
        YYs = jnp.sum(Ys * Ys, axis=1)
        XYt = lax.dot_general(xt, Yt, (((1,), (1,)), ((), ())),
                              preferred_element_type=jnp.float32)
        XYs = lax.dot_general(xs, Ys, (((1,), (1,)), ((), ())),
                              preferred_element_type=jnp.float32)
        dxy_t = XXt[:, None] - 2.0 * XYt + YYt[None, :]
        dxy_s = XXs[:, None] - 2.0 * XYs + YYs[None, :]
        sq_dxy_t = jnp.maximum(jnp.sqrt(jnp.maximum(dxy_t, 0.0)), EPS)
        sq_dxy_s = jnp.maximum(jnp.sqrt(jnp.maximum(dxy_s, 0.0)), EPS)
        for k in range(K):
            Zk = Zall[k * R:(k + 1) * R]
            ZZ = jnp.sum(Zk * Zk, axis=1)
            XZt = jnp.sum(xt * Zk, axis=1)
            XZs = jnp.sum(xs * Zk, axis=1)
            YZt = lax.dot_general(Zk, Yt, (((1,), (1,)), ((), ())),
                                  preferred_element_type=jnp.float32)
            YZs = lax.dot_general(Zk, Ys, (((1,), (1,)), ((), ())),
                                  preferred_element_type=jnp.float32)
            sq_dxz_t = jnp.maximum(jnp.sqrt(jnp.maximum(XXt - 2.0 * XZt + ZZ, 0.0)), EPS)[:, None]
            sq_dxz_s = jnp.maximum(jnp.sqrt(jnp.maximum(XXs - 2.0 * XZs + ZZ, 0.0)), EPS)[:, None]
            sq_dyz_t = jnp.maximum(jnp.sqrt(jnp.maximum(YYt[None, :] - 2.0 * YZt + ZZ[:, None], 0.0)), EPS)
            sq_dyz_s = jnp.maximum(jnp.sqrt(jnp.maximum(YYs[None, :] - 2.0 * YZs + ZZ[:, None], 0.0)), EPS)

            n1_t = YZt - XYt - XZt[:, None] + XXt[:, None]
            n1_s = YZs - XYs - XZs[:, None] + XXs[:, None]
            n2_t = XZt[:, None] - XYt - YZt + YYt[None, :]
            n2_s = XZs[:, None] - XYs - YZs + YYs[None, :]
            n3_t = XYt - XZt[:, None] - YZt + ZZ[:, None]
            n3_s = XYs - XZs[:, None] - YZs + ZZ[:, None]

            a1_t = n1_t / (sq_dxy_t * sq_dxz_t)
            a1_s = n1_s / (sq_dxy_s * sq_dxz_s)
            a2_t = n2_t / (sq_dxy_t * sq_dyz_t)
            a2_s = n2_s / (sq_dxy_s * sq_dyz_s)
            a3_t = n3_t / (sq_dxz_t * sq_dyz_t)
            a3_s = n3_s / (sq_dxz_s * sq_dyz_s)

            for at, a_s in ((a1_t, a1_s), (a2_t, a2_s), (a3_t, a3_s)):
                d = a_s - at
                ad = jnp.abs(d)
                h = jnp.where(ad <= 1.0, 0.5 * d * d, ad - 0.5)
                acc = acc + jnp.sum(h)
    ri = lax.broadcasted_iota(jnp.int32, (8, 128), 0)
    ci = lax.broadcasted_iota(jnp.int32, (8, 128), 1)
    out_ref[0] = jnp.where((ri == 0) & (ci == 0), acc, 0.0)


def _loss_call(Z, ref_t, ref_s, shared_t, shared_s):
    return pl.pallas_call(
        _loss_body,
        grid=(B,),
        in_specs=[
            pl.BlockSpec((1, K * R, D), lambda b: (b, 0, 0)),
            pl.BlockSpec((1, R, D), lambda b: (b, 0, 0)),
            pl.BlockSpec((1, R, D), lambda b: (b, 0, 0)),
            pl.BlockSpec((1, 3, S, D), lambda b: (b, 0, 0, 0)),
            pl.BlockSpec((1, 3, S, D), lambda b: (b, 0, 0, 0)),
        ],
        out_specs=pl.BlockSpec((1, 8, 128), lambda b: (b, 0, 0)),
        out_shape=jax.ShapeDtypeStruct((B, 8, 128), jnp.float32),
    )(Z, ref_t, ref_s, shared_t, shared_s)


def kernel(teacher_feats, student_feats):
    ref_perm, shared_perm = _REF_PERM, _SHARED_PERM
    ref_t = teacher_feats[:, 0, ref_perm, :]
    ref_s = student_feats[:, 0, ref_perm, :]
    shared_t = teacher_feats[:, np.array([2, 4, 6]), :, :][:, :, shared_perm, :]
    shared_s = student_feats[:, np.array([1, 2, 3]), :, :][:, :, shared_perm, :]

    idx_out = _topk_call(teacher_feats, ref_t)
    idxg = idx_out[:, :K, :].reshape(B * K * R)
    table = teacher_feats.reshape(B * 8 * P, D)
    Z = _gather_call(table, idxg).reshape(B, K * R, D)
    part = _loss_call(Z, ref_t, ref_s, shared_t, shared_s)
    total = 3 * B * R * S * K
    return part[:, 0, 0].sum() / jnp.float32(total)


# speedup vs baseline: 20.4628x; 4.2751x over previous
import functools

import numpy as np
import jax
import jax.numpy as jnp
from jax import lax
from jax.experimental import pallas as pl
from jax.experimental.pallas import tpu as pltpu
from jax.experimental.pallas import tpu_sc as plsc

B, P, D = 2, 1024, 384
EF = 4
R = 128
S = 64
K = 4
EPS = 1e-8
CCH = 256
NCH = P // CCH

TROWS = 896
SROWS = 384
NW = 32
TPW = (B * TROWS) // NW
SPW = (B * SROWS) // NW

_REF_PERM = np.array([3, 314, 747, 931, 71, 460, 167, 179, 499, 286, 843, 492, 213, 718, 787, 165, 977, 686, 441, 59, 991, 530, 744, 695, 704, 374, 829, 668, 633, 433, 932, 468, 219, 707, 0, 505, 990, 440, 41, 378, 20, 367, 247, 756, 258, 934, 602, 811, 147, 411, 461, 743, 267, 285, 997, 597, 101, 366, 791, 671, 784, 562, 279, 926, 937, 347, 981, 615, 793, 540, 1016, 185, 302, 445, 953, 1022, 312, 482, 595, 266, 731, 241, 790, 502, 453, 372, 471, 1007, 399, 551, 703, 520, 497, 728, 31, 465, 737, 473, 287, 620, 769, 776, 817, 387, 524, 331, 470, 848, 365, 839, 75, 909, 398, 857, 305, 424, 320, 1020, 292, 755, 992, 946, 952, 294, 212, 6, 939, 541], dtype=np.int32)
_SHARED_PERM = np.array([382, 452, 484, 472, 151, 773, 304, 600, 995, 278, 86, 305, 848, 836, 987, 620, 807, 637, 34, 692, 363, 486, 421, 404, 212, 794, 260, 191, 124, 128, 197, 61, 169, 546, 541, 811, 897, 855, 365, 744, 119, 104, 764, 338, 577, 832, 618, 117, 18, 430, 297, 160, 697, 172, 389, 91, 367, 914, 89, 1014, 750, 249, 560, 294], dtype=np.int32)

_ONEHOT_REF = np.zeros((R, P), dtype=np.float32)
_ONEHOT_REF[np.arange(R), _REF_PERM] = 1.0

def _static_idx():
    tconst = np.zeros((B, TROWS - 512), dtype=np.int32)
    sconst = np.zeros((B, SROWS), dtype=np.int32)
    for b in range(B):
        sh_t = np.concatenate([b * 8 * P + fr * P + _SHARED_PERM
                               for fr in (2, 4, 6)])
        tconst[b, :192] = sh_t
        tconst[b, 192:320] = b * 8 * P + _REF_PERM
        ref_s = b * 4 * P + _REF_PERM
        sh_s = np.concatenate([b * 4 * P + fr * P + _SHARED_PERM
                               for fr in (1, 2, 3)])
        sconst[b, :128] = ref_s
        sconst[b, 128:320] = sh_s
    return tconst, sconst


_TCONST, _SCONST = _static_idx()


def _topk_body(te_ref, t0_ref, oh_ref, idx_ref, xn_sc, cv_sc, ci_sc):
    b = pl.program_id(0)
    f = pl.program_id(1)

    @pl.when(f == 0)
    def _():
        F0 = t0_ref[0, 0]
        X = lax.dot_general(oh_ref[...], F0, (((1,), (0,)), ((), ())),
                            preferred_element_type=jnp.float32)
        n = jnp.sqrt(jnp.sum(X * X, axis=1, keepdims=True))
        xn_sc[...] = X / jnp.maximum(n, EPS)

    Xn = xn_sc[...]
    ms, iks = [], []
    for c in range(NCH):
        Ec = te_ref[0, 0, c * CCH:(c + 1) * CCH, :]
        en = jnp.maximum(jnp.sqrt(jnp.sum(Ec * Ec, axis=1, keepdims=True)), EPS)
        sT = lax.dot_general(Ec, Xn, (((1,), (1,)), ((), ())),
                             preferred_element_type=jnp.float32) / en
        sid = lax.broadcasted_iota(jnp.int32, (CCH, R), 0) + (f * P + c * CCH)
        for r_ in range(K):
            m = jnp.max(sT, axis=0)
            cand = jnp.where(sT == m[None, :], sid, jnp.int32(EF * P))
            ik = jnp.min(cand, axis=0)
            ms.append(m[None, :])
            iks.append(ik[None, :])
            sT = jnp.where(sid == ik[None, :], -jnp.inf, sT)
    cv_sc[pl.ds(f * NCH * K, NCH * K), :] = jnp.concatenate(ms, axis=0)
    ci_sc[pl.ds(f * NCH * K, NCH * K), :] = jnp.concatenate(iks, axis=0)

    @pl.when(f == EF - 1)
    def _():
        cv = cv_sc[...]
        ci = ci_sc[...]
        g_rows = []
        for _k in range(K):
            m = jnp.max(cv, axis=0)
            cand = jnp.where(cv == m[None, :], ci, jnp.int32(EF * P))
            ik = jnp.min(cand, axis=0)
            cv = jnp.where(ci == ik[None, :], -jnp.inf, cv)
            g = b * (8 * P) + P + ((ik >> 10) << 11) + (ik & (P - 1))
            g_rows.append(g[None, :])
        pad = jnp.zeros((8 - K, R), jnp.int32)
        idx_ref[0] = jnp.concatenate(g_rows + [pad], axis=0)


def _topk_call(teacher, onehot):
    return pl.pallas_call(
        _topk_body,
        grid=(B, EF),
        in_specs=[
            pl.BlockSpec((1, 1, P, D), lambda b, f: (b, 2 * f + 1, 0, 0)),
            pl.BlockSpec((1, 1, P, D), lambda b, f: (b, 0, 0, 0)),
            pl.BlockSpec((R, P), lambda b, f: (0, 0)),
        ],
        out_specs=pl.BlockSpec((1, 8, R), lambda b, f: (b, 0, 0)),
        out_shape=jax.ShapeDtypeStruct((B, 8, R), jnp.int32),
        scratch_shapes=[
            pltpu.VMEM((R, D), jnp.float32),
            pltpu.VMEM((EF * NCH * K, R), jnp.float32),
            pltpu.VMEM((EF * NCH * K, R), jnp.int32),
        ],
    )(teacher, teacher, onehot)


def _gather_body(ttab_hbm, stab_hbm, idxt_hbm, idxs_hbm, outt_hbm, outs_hbm,
                 idxt_v, rowst_v, idxs_v, rowss_v, sem):
    wid = lax.axis_index("s") * 2 + lax.axis_index("c")
    baset = wid * TPW
    pltpu.sync_copy(idxt_hbm.at[pl.ds(baset, TPW)], idxt_v)
    pltpu.async_copy(ttab_hbm.at[idxt_v], rowst_v, sem).wait()
    pltpu.sync_copy(rowst_v, outt_hbm.at[pl.ds(baset, TPW)])
    bases = wid * SPW
    pltpu.sync_copy(idxs_hbm.at[pl.ds(bases, SPW)], idxs_v)
    pltpu.async_copy(stab_hbm.at[idxs_v], rowss_v, sem).wait()
    pltpu.sync_copy(rowss_v, outs_hbm.at[pl.ds(bases, SPW)])


def _gather_call(ttab, stab, idx_t, idx_s):
    mesh = plsc.VectorSubcoreMesh(core_axis_name="c", subcore_axis_name="s")
    k = functools.partial(
        pl.kernel,
        out_type=(jax.ShapeDtypeStruct((B * TROWS, D), jnp.float32),
                  jax.ShapeDtypeStruct((B * SROWS, D), jnp.float32)),
        mesh=mesh,
        scratch_types=[
            pltpu.VMEM((TPW,), jnp.int32),
            pltpu.VMEM((TPW, D), jnp.float32),
            pltpu.VMEM((SPW,), jnp.int32),
            pltpu.VMEM((SPW, D), jnp.float32),
            pltpu.SemaphoreType.DMA,
        ],
    )(_gather_body)
    return k(ttab, stab, idx_t, idx_s)


def _loss_body(t_ref, s_ref, out_ref):
    ones = jnp.ones((1, D), jnp.float32)
    acc = jnp.zeros((R, S), jnp.float32)
    xt = t_ref[0, 704:832, :]
    xs = s_ref[0, 0:128, :]
    XXt = jnp.sum(xt * xt, axis=1, keepdims=True)
    XXs = jnp.sum(xs * xs, axis=1, keepdims=True)
    pre = []
    for f in range(3):
        Yt = t_ref[0, 512 + f * S:512 + (f + 1) * S, :]
        Ys = s_ref[0, 128 + f * S:128 + (f + 1) * S, :]
        YYt = lax.dot_general(ones, Yt * Yt, (((1,), (1,)), ((), ())),
                              preferred_element_type=jnp.float32)
        YYs = lax.dot_general(ones, Ys * Ys, (((1,), (1,)), ((), ())),
                              preferred_element_type=jnp.float32)
        XYt = lax.dot_general(xt, Yt, (((1,), (1,)), ((), ())),
                              preferred_element_type=jnp.float32)
        XYs = lax.dot_general(xs, Ys, (((1,), (1,)), ((), ())),
                              preferred_element_type=jnp.float32)
        dxy_t = XXt - 2.0 * XYt + YYt
        dxy_s = XXs - 2.0 * XYs + YYs
        sq_dxy_t = jnp.maximum(jnp.sqrt(jnp.maximum(dxy_t, 0.0)), EPS)
        sq_dxy_s = jnp.maximum(jnp.sqrt(jnp.maximum(dxy_s, 0.0)), EPS)
        pre.append((YYt, YYs, XYt, XYs, sq_dxy_t, sq_dxy_s))
    for k in range(K):
        Zk = t_ref[0, k * R:(k + 1) * R, :]
        ZZ = jnp.sum(Zk * Zk, axis=1, keepdims=True)
        XZt = jnp.sum(xt * Zk, axis=1, keepdims=True)
        XZs = jnp.sum(xs * Zk, axis=1, keepdims=True)
        sq_dxz_t = jnp.maximum(jnp.sqrt(jnp.maximum(XXt - 2.0 * XZt + ZZ, 0.0)), EPS)
        sq_dxz_s = jnp.maximum(jnp.sqrt(jnp.maximum(XXs - 2.0 * XZs + ZZ, 0.0)), EPS)
        for f in range(3):
            YYt, YYs, XYt, XYs, sq_dxy_t, sq_dxy_s = pre[f]
            Yt = t_ref[0, 512 + f * S:512 + (f + 1) * S, :]
            Ys = s_ref[0, 128 + f * S:128 + (f + 1) * S, :]
            YZt = lax.dot_general(Zk, Yt, (((1,), (1,)), ((), ())),
                                  preferred_element_type=jnp.float32)
            YZs = lax.dot_general(Zk, Ys, (((1,), (1,)), ((), ())),
                                  preferred_element_type=jnp.float32)
            sq_dyz_t = jnp.maximum(jnp.sqrt(jnp.maximum(YYt - 2.0 * YZt + ZZ, 0.0)), EPS)
            sq_dyz_s = jnp.maximum(jnp.sqrt(jnp.maximum(YYs - 2.0 * YZs + ZZ, 0.0)), EPS)

            a1_t = (YZt - XYt - XZt + XXt) / (sq_dxy_t * sq_dxz_t)
            a1_s = (YZs - XYs - XZs + XXs) / (sq_dxy_s * sq_dxz_s)
            a2_t = (XZt - XYt - YZt + YYt) / (sq_dxy_t * sq_dyz_t)
            a2_s = (XZs - XYs - YZs + YYs) / (sq_dxy_s * sq_dyz_s)
            a3_t = (XYt - XZt - YZt + ZZ) / (sq_dxz_t * sq_dyz_t)
            a3_s = (XYs - XZs - YZs + ZZ) / (sq_dxz_s * sq_dyz_s)

            for at, a_s in ((a1_t, a1_s), (a2_t, a2_s), (a3_t, a3_s)):
                d = a_s - at
                ad = jnp.abs(d)
                acc = acc + jnp.where(ad <= 1.0, 0.5 * d * d, ad - 0.5)
    tot = jnp.sum(acc)
    ri = lax.broadcasted_iota(jnp.int32, (8, 128), 0)
    ci = lax.broadcasted_iota(jnp.int32, (8, 128), 1)
    out_ref[0] = jnp.where((ri == 0) & (ci == 0), tot, 0.0)


def _loss_call(rows_t, rows_s):
    return pl.pallas_call(
        _loss_body,
        grid=(B,),
        in_specs=[
            pl.BlockSpec((1, TROWS, D), lambda b: (b, 0, 0)),
            pl.BlockSpec((1, SROWS, D), lambda b: (b, 0, 0)),
        ],
        out_specs=pl.BlockSpec((1, 8, 128), lambda b: (b, 0, 0)),
        out_shape=jax.ShapeDtypeStruct((B, 8, 128), jnp.float32),
    )(rows_t, rows_s)


def kernel(teacher_feats, student_feats):
    onehot = jnp.asarray(_ONEHOT_REF)
    idx_out = _topk_call(teacher_feats, onehot)
    zidx = idx_out[:, :K, :].reshape(B, K * R)
    idx_t = jnp.concatenate([zidx, jnp.asarray(_TCONST)], axis=1).reshape(B * TROWS)
    idx_s = jnp.asarray(_SCONST).reshape(B * SROWS)
    ttab = teacher_feats.reshape(B * 8 * P, D)
    stab = student_feats.reshape(B * 4 * P, D)
    rows_t, rows_s = _gather_call(ttab, stab, idx_t, idx_s)
    part = _loss_call(rows_t.reshape(B, TROWS, D), rows_s.reshape(B, SROWS, D))
    total = 3 * B * R * S * K
    return part[:, 0, 0].sum() / jnp.float32(total)


# speedup vs baseline: 20.9814x; 1.0253x over previous
import functools

import numpy as np
import jax
import jax.numpy as jnp
from jax import lax
from jax.experimental import pallas as pl
from jax.experimental.pallas import tpu as pltpu
from jax.experimental.pallas import tpu_sc as plsc

B, P, D = 2, 1024, 384
EF = 4
R = 128
S = 64
K = 4
EPS = 1e-8
CCH = 256
NCH = P // CCH

TROWS = 384
SROWS = 384
NW = 32
TPW = (B * TROWS) // NW
SPW = (B * SROWS) // NW
ZPW = (B * K * R) // NW

_REF_PERM = np.array([3, 314, 747, 931, 71, 460, 167, 179, 499, 286, 843, 492, 213, 718, 787, 165, 977, 686, 441, 59, 991, 530, 744, 695, 704, 374, 829, 668, 633, 433, 932, 468, 219, 707, 0, 505, 990, 440, 41, 378, 20, 367, 247, 756, 258, 934, 602, 811, 147, 411, 461, 743, 267, 285, 997, 597, 101, 366, 791, 671, 784, 562, 279, 926, 937, 347, 981, 615, 793, 540, 1016, 185, 302, 445, 953, 1022, 312, 482, 595, 266, 731, 241, 790, 502, 453, 372, 471, 1007, 399, 551, 703, 520, 497, 728, 31, 465, 737, 473, 287, 620, 769, 776, 817, 387, 524, 331, 470, 848, 365, 839, 75, 909, 398, 857, 305, 424, 320, 1020, 292, 755, 992, 946, 952, 294, 212, 6, 939, 541], dtype=np.int32)
_SHARED_PERM = np.array([382, 452, 484, 472, 151, 773, 304, 600, 995, 278, 86, 305, 848, 836, 987, 620, 807, 637, 34, 692, 363, 486, 421, 404, 212, 794, 260, 191, 124, 128, 197, 61, 169, 546, 541, 811, 897, 855, 365, 744, 119, 104, 764, 338, 577, 832, 618, 117, 18, 430, 297, 160, 697, 172, 389, 91, 367, 914, 89, 1014, 750, 249, 560, 294], dtype=np.int32)

_ONEHOT_REF = np.zeros((R, P), dtype=np.float32)
_ONEHOT_REF[np.arange(R), _REF_PERM] = 1.0

def _static_idx():
    tconst = np.zeros((B, TROWS), dtype=np.int32)
    sconst = np.zeros((B, SROWS), dtype=np.int32)
    for b in range(B):
        sh_t = np.concatenate([b * 8 * P + fr * P + _SHARED_PERM
                               for fr in (2, 4, 6)])
        tconst[b, :192] = sh_t
        tconst[b, 192:320] = b * 8 * P + _REF_PERM
        ref_s = b * 4 * P + _REF_PERM
        sh_s = np.concatenate([b * 4 * P + fr * P + _SHARED_PERM
                               for fr in (1, 2, 3)])
        sconst[b, :128] = ref_s
        sconst[b, 128:320] = sh_s
    return tconst, sconst


_TCONST, _SCONST = _static_idx()


def _topk_body(te_ref, t0_ref, oh_ref, idx_ref, xn_sc, cv_sc, ci_sc):
    b = pl.program_id(0)
    f = pl.program_id(1)

    @pl.when(f == 0)
    def _():
        F0 = t0_ref[0, 0]
        X = lax.dot_general(oh_ref[...], F0, (((1,), (0,)), ((), ())),
                            preferred_element_type=jnp.float32)
        n = jnp.sqrt(jnp.sum(X * X, axis=1, keepdims=True))
        xn_sc[...] = X / jnp.maximum(n, EPS)

    Xn = xn_sc[...]
    ms, iks = [], []
    for c in range(NCH):
        Ec = te_ref[0, 0, c * CCH:(c + 1) * CCH, :]
        en = jnp.maximum(jnp.sqrt(jnp.sum(Ec * Ec, axis=1, keepdims=True)), EPS)
        sT = lax.dot_general(Ec, Xn, (((1,), (1,)), ((), ())),
                             preferred_element_type=jnp.float32) / en
        sid = lax.broadcasted_iota(jnp.int32, (CCH, R), 0) + (f * P + c * CCH)
        for r_ in range(K):
            m = jnp.max(sT, axis=0)
            cand = jnp.where(sT == m[None, :], sid, jnp.int32(EF * P))
            ik = jnp.min(cand, axis=0)
            ms.append(m[None, :])
            iks.append(ik[None, :])
            sT = jnp.where(sid == ik[None, :], -jnp.inf, sT)
    cv_sc[pl.ds(f * NCH * K, NCH * K), :] = jnp.concatenate(ms, axis=0)
    ci_sc[pl.ds(f * NCH * K, NCH * K), :] = jnp.concatenate(iks, axis=0)

    @pl.when(f == EF - 1)
    def _():
        cv = cv_sc[...]
        ci = ci_sc[...]
        g_rows = []
        for _k in range(K):
            m = jnp.max(cv, axis=0)
            cand = jnp.where(cv == m[None, :], ci, jnp.int32(EF * P))
            ik = jnp.min(cand, axis=0)
            cv = jnp.where(ci == ik[None, :], -jnp.inf, cv)
            g = b * (8 * P) + P + ((ik >> 10) << 11) + (ik & (P - 1))
            g_rows.append(g[None, :])
        pad = jnp.zeros((8 - K, R), jnp.int32)
        idx_ref[0] = jnp.concatenate(g_rows + [pad], axis=0)


def _topk_call(teacher, onehot):
    return pl.pallas_call(
        _topk_body,
        grid=(B, EF),
        in_specs=[
            pl.BlockSpec((1, 1, P, D), lambda b, f: (b, 2 * f + 1, 0, 0)),
            pl.BlockSpec((1, 1, P, D), lambda b, f: (b, 0, 0, 0)),
            pl.BlockSpec((R, P), lambda b, f: (0, 0)),
        ],
        out_specs=pl.BlockSpec((1, 8, R), lambda b, f: (b, 0, 0)),
        out_shape=jax.ShapeDtypeStruct((B, 8, R), jnp.int32),
        scratch_shapes=[
            pltpu.VMEM((R, D), jnp.float32),
            pltpu.VMEM((EF * NCH * K, R), jnp.float32),
            pltpu.VMEM((EF * NCH * K, R), jnp.int32),
        ],
    )(teacher, teacher, onehot)


def _static_gather_body(ttab_hbm, stab_hbm, idxt_hbm, idxs_hbm,
                        outt_hbm, outs_hbm,
                        idxt_v, rowst_v, idxs_v, rowss_v, semt, sems):
    wid = lax.axis_index("s") * 2 + lax.axis_index("c")
    baset = wid * TPW
    bases = wid * SPW
    pltpu.sync_copy(idxt_hbm.at[pl.ds(baset, TPW)], idxt_v)
    pltpu.sync_copy(idxs_hbm.at[pl.ds(bases, SPW)], idxs_v)
    ct = pltpu.async_copy(ttab_hbm.at[idxt_v], rowst_v, semt)
    cs = pltpu.async_copy(stab_hbm.at[idxs_v], rowss_v, sems)
    ct.wait()
    wt = pltpu.async_copy(rowst_v, outt_hbm.at[pl.ds(baset, TPW)], semt)
    cs.wait()
    ws = pltpu.async_copy(rowss_v, outs_hbm.at[pl.ds(bases, SPW)], sems)
    wt.wait()
    ws.wait()


def _static_gather_call(ttab, stab, idx_t, idx_s):
    mesh = plsc.VectorSubcoreMesh(core_axis_name="c", subcore_axis_name="s")
    k = functools.partial(
        pl.kernel,
        out_type=(jax.ShapeDtypeStruct((B * TROWS, D), jnp.float32),
                  jax.ShapeDtypeStruct((B * SROWS, D), jnp.float32)),
        mesh=mesh,
        scratch_types=[
            pltpu.VMEM((TPW,), jnp.int32),
            pltpu.VMEM((TPW, D), jnp.float32),
            pltpu.VMEM((SPW,), jnp.int32),
            pltpu.VMEM((SPW, D), jnp.float32),
            pltpu.SemaphoreType.DMA,
            pltpu.SemaphoreType.DMA,
        ],
    )(_static_gather_body)
    return k(ttab, stab, idx_t, idx_s)


def _z_gather_body(ttab_hbm, idx_hbm, out_hbm, idx_v, rows_v, sem):
    wid = lax.axis_index("s") * 2 + lax.axis_index("c")
    off = wid * ZPW
    src = off + (off // 512) * 512
    pltpu.sync_copy(idx_hbm.at[pl.ds(src, ZPW)], idx_v)
    pltpu.async_copy(ttab_hbm.at[idx_v], rows_v, sem).wait()
    pltpu.sync_copy(rows_v, out_hbm.at[pl.ds(off, ZPW)])


def _z_gather_call(ttab, idx_flat):
    mesh = plsc.VectorSubcoreMesh(core_axis_name="c", subcore_axis_name="s")
    k = functools.partial(
        pl.kernel,
        out_type=jax.ShapeDtypeStruct((B * K * R, D), jnp.float32),
        mesh=mesh,
        scratch_types=[
            pltpu.VMEM((ZPW,), jnp.int32),
            pltpu.VMEM((ZPW, D), jnp.float32),
            pltpu.SemaphoreType.DMA,
        ],
    )(_z_gather_body)
    return k(ttab, idx_flat)


def _loss_body(z_ref, t_ref, s_ref, out_ref):
    ones = jnp.ones((1, D), jnp.float32)
    acc = jnp.zeros((R, S), jnp.float32)
    xt = t_ref[0, 192:320, :]
    xs = s_ref[0, 0:128, :]
    XXt = jnp.sum(xt * xt, axis=1, keepdims=True)
    XXs = jnp.sum(xs * xs, axis=1, keepdims=True)
    pre = []
    for f in range(3):
        Yt = t_ref[0, f * S:(f + 1) * S, :]
        Ys = s_ref[0, 128 + f * S:128 + (f + 1) * S, :]
        YYt = lax.dot_general(ones, Yt * Yt, (((1,), (1,)), ((), ())),
                              preferred_element_type=jnp.float32)
        YYs = lax.dot_general(ones, Ys * Ys, (((1,), (1,)), ((), ())),
                              preferred_element_type=jnp.float32)
        XYt = lax.dot_general(xt, Yt, (((1,), (1,)), ((), ())),
                              preferred_element_type=jnp.float32)
        XYs = lax.dot_general(xs, Ys, (((1,), (1,)), ((), ())),
                              preferred_element_type=jnp.float32)
        dxy_t = XXt - 2.0 * XYt + YYt
        dxy_s = XXs - 2.0 * XYs + YYs
        sq_dxy_t = jnp.maximum(jnp.sqrt(jnp.maximum(dxy_t, 0.0)), EPS)
        sq_dxy_s = jnp.maximum(jnp.sqrt(jnp.maximum(dxy_s, 0.0)), EPS)
        pre.append((YYt, YYs, XYt, XYs, sq_dxy_t, sq_dxy_s))
    for k in range(K):
        Zk = z_ref[0, k * R:(k + 1) * R, :]
        ZZ = jnp.sum(Zk * Zk, axis=1, keepdims=True)
        XZt = jnp.sum(xt * Zk, axis=1, keepdims=True)
        XZs = jnp.sum(xs * Zk, axis=1, keepdims=True)
        sq_dxz_t = jnp.maximum(jnp.sqrt(jnp.maximum(XXt - 2.0 * XZt + ZZ, 0.0)), EPS)
        sq_dxz_s = jnp.maximum(jnp.sqrt(jnp.maximum(XXs - 2.0 * XZs + ZZ, 0.0)), EPS)
        for f in range(3):
            YYt, YYs, XYt, XYs, sq_dxy_t, sq_dxy_s = pre[f]
            Yt = t_ref[0, f * S:(f + 1) * S, :]
            Ys = s_ref[0, 128 + f * S:128 + (f + 1) * S, :]
            YZt = lax.dot_general(Zk, Yt, (((1,), (1,)), ((), ())),
                                  preferred_element_type=jnp.float32)
            YZs = lax.dot_general(Zk, Ys, (((1,), (1,)), ((), ())),
                                  preferred_element_type=jnp.float32)
            sq_dyz_t = jnp.maximum(jnp.sqrt(jnp.maximum(YYt - 2.0 * YZt + ZZ, 0.0)), EPS)
            sq_dyz_s = jnp.maximum(jnp.sqrt(jnp.maximum(YYs - 2.0 * YZs + ZZ, 0.0)), EPS)

            a1_t = (YZt - XYt - XZt + XXt) / (sq_dxy_t * sq_dxz_t)
            a1_s = (YZs - XYs - XZs + XXs) / (sq_dxy_s * sq_dxz_s)
            a2_t = (XZt - XYt - YZt + YYt) / (sq_dxy_t * sq_dyz_t)
            a2_s = (XZs - XYs - YZs + YYs) / (sq_dxy_s * sq_dyz_s)
            a3_t = (XYt - XZt - YZt + ZZ) / (sq_dxz_t * sq_dyz_t)
            a3_s = (XYs - XZs - YZs + ZZ) / (sq_dxz_s * sq_dyz_s)

            for at, a_s in ((a1_t, a1_s), (a2_t, a2_s), (a3_t, a3_s)):
                d = a_s - at
                ad = jnp.abs(d)
                acc = acc + jnp.where(ad <= 1.0, 0.5 * d * d, ad - 0.5)
    tot = jnp.sum(acc)
    ri = lax.broadcasted_iota(jnp.int32, (8, 128), 0)
    ci = lax.broadcasted_iota(jnp.int32, (8, 128), 1)
    scale = 1.0 / np.float32(3 * B * R * S * K)
    part = jnp.where((ri == 0) & (ci == 0), tot * scale, 0.0)

    @pl.when(pl.program_id(0) == 0)
    def _():
        out_ref[0] = part

    @pl.when(pl.program_id(0) != 0)
    def _():
        out_ref[0] = out_ref[0] + part


def _loss_call(rows_z, rows_t, rows_s):
    return pl.pallas_call(
        _loss_body,
        grid=(B,),
        in_specs=[
            pl.BlockSpec((1, K * R, D), lambda b: (b, 0, 0)),
            pl.BlockSpec((1, TROWS, D), lambda b: (b, 0, 0)),
            pl.BlockSpec((1, SROWS, D), lambda b: (b, 0, 0)),
        ],
        out_specs=pl.BlockSpec((1, 8, 128), lambda b: (0, 0, 0)),
        out_shape=jax.ShapeDtypeStruct((1, 8, 128), jnp.float32),
    )(rows_z, rows_t, rows_s)


def kernel(teacher_feats, student_feats):
    onehot = jnp.asarray(_ONEHOT_REF)
    ttab = teacher_feats.reshape(B * 8 * P, D)
    stab = student_feats.reshape(B * 4 * P, D)
    idx_t = jnp.asarray(_TCONST).reshape(B * TROWS)
    idx_s = jnp.asarray(_SCONST).reshape(B * SROWS)
    rows_t, rows_s = _static_gather_call(ttab, stab, idx_t, idx_s)
    idx_out = _topk_call(teacher_feats, onehot)
    rows_z = _z_gather_call(ttab, idx_out.reshape(B * 8 * R))
    part = _loss_call(rows_z.reshape(B, K * R, D),
                      rows_t.reshape(B, TROWS, D),
                      rows_s.reshape(B, SROWS, D))
    return part[0, 0, 0]


# speedup vs baseline: 21.0575x; 1.0036x over previous
import functools

import numpy as np
import jax
import jax.numpy as jnp
from jax import lax
from jax.experimental import pallas as pl
from jax.experimental.pallas import tpu as pltpu
from jax.experimental.pallas import tpu_sc as plsc

B, P, D = 2, 1024, 384
EF = 4
R = 128
S = 64
K = 4
EPS = 1e-8
CCH = 256
NCH = P // CCH

TROWS = 384
SROWS = 384
NW = 32
TPW = (B * TROWS) // NW
SPW = (B * SROWS) // NW
ZPW = (B * K * R) // NW

_REF_PERM = np.array([3, 314, 747, 931, 71, 460, 167, 179, 499, 286, 843, 492, 213, 718, 787, 165, 977, 686, 441, 59, 991, 530, 744, 695, 704, 374, 829, 668, 633, 433, 932, 468, 219, 707, 0, 505, 990, 440, 41, 378, 20, 367, 247, 756, 258, 934, 602, 811, 147, 411, 461, 743, 267, 285, 997, 597, 101, 366, 791, 671, 784, 562, 279, 926, 937, 347, 981, 615, 793, 540, 1016, 185, 302, 445, 953, 1022, 312, 482, 595, 266, 731, 241, 790, 502, 453, 372, 471, 1007, 399, 551, 703, 520, 497, 728, 31, 465, 737, 473, 287, 620, 769, 776, 817, 387, 524, 331, 470, 848, 365, 839, 75, 909, 398, 857, 305, 424, 320, 1020, 292, 755, 992, 946, 952, 294, 212, 6, 939, 541], dtype=np.int32)
_SHARED_PERM = np.array([382, 452, 484, 472, 151, 773, 304, 600, 995, 278, 86, 305, 848, 836, 987, 620, 807, 637, 34, 692, 363, 486, 421, 404, 212, 794, 260, 191, 124, 128, 197, 61, 169, 546, 541, 811, 897, 855, 365, 744, 119, 104, 764, 338, 577, 832, 618, 117, 18, 430, 297, 160, 697, 172, 389, 91, 367, 914, 89, 1014, 750, 249, 560, 294], dtype=np.int32)

_ONEHOT_REF = np.zeros((R, P), dtype=np.float32)
_ONEHOT_REF[np.arange(R), _REF_PERM] = 1.0

def _static_idx():
    tconst = np.zeros((B, TROWS), dtype=np.int32)
    sconst = np.zeros((B, SROWS), dtype=np.int32)
    for b in range(B):
        sh_t = np.concatenate([b * 8 * P + fr * P + _SHARED_PERM
                               for fr in (2, 4, 6)])
        tconst[b, :192] = sh_t
        tconst[b, 192:320] = b * 8 * P + _REF_PERM
        ref_s = b * 4 * P + _REF_PERM
        sh_s = np.concatenate([b * 4 * P + fr * P + _SHARED_PERM
                               for fr in (1, 2, 3)])
        sconst[b, :128] = ref_s
        sconst[b, 128:320] = sh_s
    return tconst, sconst


_TCONST, _SCONST = _static_idx()


def _topk_body(te_ref, t0_ref, oh_ref, idx_ref, xn_sc, cv_sc, ci_sc):
    b = pl.program_id(0)
    f = pl.program_id(1)

    @pl.when(f == 0)
    def _():
        F0 = t0_ref[0, 0]
        X = lax.dot_general(oh_ref[...], F0, (((1,), (0,)), ((), ())),
                            preferred_element_type=jnp.float32)
        n = jnp.sqrt(jnp.sum(X * X, axis=1, keepdims=True))
        xn_sc[...] = X / jnp.maximum(n, EPS)

    Xn = xn_sc[...]
    ms, iks = [], []
    for c in range(NCH):
        Ec = te_ref[0, 0, c * CCH:(c + 1) * CCH, :]
        en = jnp.maximum(jnp.sqrt(jnp.sum(Ec * Ec, axis=1, keepdims=True)), EPS)
        sT = lax.dot_general(Ec, Xn, (((1,), (1,)), ((), ())),
                             preferred_element_type=jnp.float32) / en
        sid = lax.broadcasted_iota(jnp.int32, (CCH, R), 0) + (f * P + c * CCH)
        for r_ in range(K):
            m = jnp.max(sT, axis=0)
            cand = jnp.where(sT == m[None, :], sid, jnp.int32(EF * P))
            ik = jnp.min(cand, axis=0)
            ms.append(m[None, :])
            iks.append(ik[None, :])
            sT = jnp.where(sid == ik[None, :], -jnp.inf, sT)
    cv_sc[pl.ds(f * NCH * K, NCH * K), :] = jnp.concatenate(ms, axis=0)
    ci_sc[pl.ds(f * NCH * K, NCH * K), :] = jnp.concatenate(iks, axis=0)

    @pl.when(f == EF - 1)
    def _():
        cv = cv_sc[...]
        ci = ci_sc[...]
        g_rows = []
        for _k in range(K):
            m = jnp.max(cv, axis=0)
            cand = jnp.where(cv == m[None, :], ci, jnp.int32(EF * P))
            ik = jnp.min(cand, axis=0)
            cv = jnp.where(ci == ik[None, :], -jnp.inf, cv)
            g = b * (8 * P) + P + ((ik >> 10) << 11) + (ik & (P - 1))
            g_rows.append(g[None, :])
        pad = jnp.zeros((8 - K, R), jnp.int32)
        idx_ref[0] = jnp.concatenate(g_rows + [pad], axis=0)


def _topk_call(teacher, onehot):
    return pl.pallas_call(
        _topk_body,
        grid=(B, EF),
        in_specs=[
            pl.BlockSpec((1, 1, P, D), lambda b, f: (b, 2 * f + 1, 0, 0)),
            pl.BlockSpec((1, 1, P, D), lambda b, f: (b, 0, 0, 0)),
            pl.BlockSpec((R, P), lambda b, f: (0, 0)),
        ],
        out_specs=pl.BlockSpec((1, 8, R), lambda b, f: (b, 0, 0)),
        out_shape=jax.ShapeDtypeStruct((B, 8, R), jnp.int32),
        scratch_shapes=[
            pltpu.VMEM((R, D), jnp.float32),
            pltpu.VMEM((EF * NCH * K, R), jnp.float32),
            pltpu.VMEM((EF * NCH * K, R), jnp.int32),
        ],
    )(teacher, teacher, onehot)


def _gather_body(ttab_hbm, stab_hbm, idxz_hbm, idxt_hbm, idxs_hbm,
                 outz_hbm, outt_hbm, outs_hbm,
                 idxz_v, rowsz_v, idxt_v, rowst_v, idxs_v, rowss_v,
                 semz, semt, sems):
    wid = lax.axis_index("s") * 2 + lax.axis_index("c")
    basez = wid * ZPW
    baset = wid * TPW
    bases = wid * SPW
    pltpu.sync_copy(idxz_hbm.at[pl.ds(basez, ZPW)], idxz_v)
    pltpu.sync_copy(idxt_hbm.at[pl.ds(baset, TPW)], idxt_v)
    pltpu.sync_copy(idxs_hbm.at[pl.ds(bases, SPW)], idxs_v)
    cz = pltpu.async_copy(ttab_hbm.at[idxz_v], rowsz_v, semz)
    ct = pltpu.async_copy(ttab_hbm.at[idxt_v], rowst_v, semt)
    cs = pltpu.async_copy(stab_hbm.at[idxs_v], rowss_v, sems)
    cz.wait()
    wz = pltpu.async_copy(rowsz_v, outz_hbm.at[pl.ds(basez, ZPW)], semz)
    ct.wait()
    wt = pltpu.async_copy(rowst_v, outt_hbm.at[pl.ds(baset, TPW)], semt)
    cs.wait()
    ws = pltpu.async_copy(rowss_v, outs_hbm.at[pl.ds(bases, SPW)], sems)
    wz.wait()
    wt.wait()
    ws.wait()


def _gather_call(ttab, stab, idx_z, idx_t, idx_s):
    mesh = plsc.VectorSubcoreMesh(core_axis_name="c", subcore_axis_name="s")
    k = functools.partial(
        pl.kernel,
        out_type=(jax.ShapeDtypeStruct((B * K * R, D), jnp.float32),
                  jax.ShapeDtypeStruct((B * TROWS, D), jnp.float32),
                  jax.ShapeDtypeStruct((B * SROWS, D), jnp.float32)),
        mesh=mesh,
        scratch_types=[
            pltpu.VMEM((ZPW,), jnp.int32),
            pltpu.VMEM((ZPW, D), jnp.float32),
            pltpu.VMEM((TPW,), jnp.int32),
            pltpu.VMEM((TPW, D), jnp.float32),
            pltpu.VMEM((SPW,), jnp.int32),
            pltpu.VMEM((SPW, D), jnp.float32),
            pltpu.SemaphoreType.DMA,
            pltpu.SemaphoreType.DMA,
            pltpu.SemaphoreType.DMA,
        ],
    )(_gather_body)
    return k(ttab, stab, idx_z, idx_t, idx_s)


def _loss_body(z_ref, t_ref, s_ref, out_ref):
    ones = jnp.ones((1, D), jnp.float32)
    acc = jnp.zeros((R, S), jnp.float32)
    xt = t_ref[0, 192:320, :]
    xs = s_ref[0, 0:128, :]
    XXt = jnp.sum(xt * xt, axis=1, keepdims=True)
    XXs = jnp.sum(xs * xs, axis=1, keepdims=True)
    pre = []
    for f in range(3):
        Yt = t_ref[0, f * S:(f + 1) * S, :]
        Ys = s_ref[0, 128 + f * S:128 + (f + 1) * S, :]
        YYt = lax.dot_general(ones, Yt * Yt, (((1,), (1,)), ((), ())),
                              preferred_element_type=jnp.float32)
        YYs = lax.dot_general(ones, Ys * Ys, (((1,), (1,)), ((), ())),
                              preferred_element_type=jnp.float32)
        XYt = lax.dot_general(xt, Yt, (((1,), (1,)), ((), ())),
                              preferred_element_type=jnp.float32)
        XYs = lax.dot_general(xs, Ys, (((1,), (1,)), ((), ())),
                              preferred_element_type=jnp.float32)
        dxy_t = XXt - 2.0 * XYt + YYt
        dxy_s = XXs - 2.0 * XYs + YYs
        sq_dxy_t = jnp.maximum(jnp.sqrt(jnp.maximum(dxy_t, 0.0)), EPS)
        sq_dxy_s = jnp.maximum(jnp.sqrt(jnp.maximum(dxy_s, 0.0)), EPS)
        pre.append((YYt, YYs, XYt, XYs, sq_dxy_t, sq_dxy_s))
    for k in range(K):
        Zk = z_ref[0, k * R:(k + 1) * R, :]
        ZZ = jnp.sum(Zk * Zk, axis=1, keepdims=True)
        XZt = jnp.sum(xt * Zk, axis=1, keepdims=True)
        XZs = jnp.sum(xs * Zk, axis=1, keepdims=True)
        sq_dxz_t = jnp.maximum(jnp.sqrt(jnp.maximum(XXt - 2.0 * XZt + ZZ, 0.0)), EPS)
        sq_dxz_s = jnp.maximum(jnp.sqrt(jnp.maximum(XXs - 2.0 * XZs + ZZ, 0.0)), EPS)
        for f in range(3):
            YYt, YYs, XYt, XYs, sq_dxy_t, sq_dxy_s = pre[f]
            Yt = t_ref[0, f * S:(f + 1) * S, :]
            Ys = s_ref[0, 128 + f * S:128 + (f + 1) * S, :]
            YZt = lax.dot_general(Zk, Yt, (((1,), (1,)), ((), ())),
                                  preferred_element_type=jnp.float32)
            YZs = lax.dot_general(Zk, Ys, (((1,), (1,)), ((), ())),
                                  preferred_element_type=jnp.float32)
            sq_dyz_t = jnp.maximum(jnp.sqrt(jnp.maximum(YYt - 2.0 * YZt + ZZ, 0.0)), EPS)
            sq_dyz_s = jnp.maximum(jnp.sqrt(jnp.maximum(YYs - 2.0 * YZs + ZZ, 0.0)), EPS)

            a1_t = (YZt - XYt - XZt + XXt) / (sq_dxy_t * sq_dxz_t)
            a1_s = (YZs - XYs - XZs + XXs) / (sq_dxy_s * sq_dxz_s)
            a2_t = (XZt - XYt - YZt + YYt) / (sq_dxy_t * sq_dyz_t)
            a2_s = (XZs - XYs - YZs + YYs) / (sq_dxy_s * sq_dyz_s)
            a3_t = (XYt - XZt - YZt + ZZ) / (sq_dxz_t * sq_dyz_t)
            a3_s = (XYs - XZs - YZs + ZZ) / (sq_dxz_s * sq_dyz_s)

            for at, a_s in ((a1_t, a1_s), (a2_t, a2_s), (a3_t, a3_s)):
                d = a_s - at
                ad = jnp.abs(d)
                acc = acc + jnp.where(ad <= 1.0, 0.5 * d * d, ad - 0.5)
    tot = jnp.sum(acc)
    ri = lax.broadcasted_iota(jnp.int32, (8, 128), 0)
    ci = lax.broadcasted_iota(jnp.int32, (8, 128), 1)
    scale = 1.0 / np.float32(3 * B * R * S * K)
    part = jnp.where((ri == 0) & (ci == 0), tot * scale, 0.0)

    @pl.when(pl.program_id(0) == 0)
    def _():
        out_ref[0] = part

    @pl.when(pl.program_id(0) != 0)
    def _():
        out_ref[0] = out_ref[0] + part


def _loss_call(rows_z, rows_t, rows_s):
    return pl.pallas_call(
        _loss_body,
        grid=(B,),
        in_specs=[
            pl.BlockSpec((1, K * R, D), lambda b: (b, 0, 0)),
            pl.BlockSpec((1, TROWS, D), lambda b: (b, 0, 0)),
            pl.BlockSpec((1, SROWS, D), lambda b: (b, 0, 0)),
        ],
        out_specs=pl.BlockSpec((1, 8, 128), lambda b: (0, 0, 0)),
        out_shape=jax.ShapeDtypeStruct((1, 8, 128), jnp.float32),
    )(rows_z, rows_t, rows_s)


def kernel(teacher_feats, student_feats):
    onehot = jnp.asarray(_ONEHOT_REF)
    ttab = teacher_feats.reshape(B * 8 * P, D)
    stab = student_feats.reshape(B * 4 * P, D)
    idx_t = jnp.asarray(_TCONST).reshape(B * TROWS)
    idx_s = jnp.asarray(_SCONST).reshape(B * SROWS)
    idx_out = _topk_call(teacher_feats, onehot)
    idx_z = idx_out[:, :K, :].reshape(B * K * R)
    rows_z, rows_t, rows_s = _gather_call(ttab, stab, idx_z, idx_t, idx_s)
    part = _loss_call(rows_z.reshape(B, K * R, D),
                      rows_t.reshape(B, TROWS, D),
                      rows_s.reshape(B, SROWS, D))
    return part[0, 0, 0]


# speedup vs baseline: 23.7735x; 1.1290x over previous
import functools

import numpy as np
import jax
import jax.numpy as jnp
from jax import lax
from jax.experimental import pallas as pl
from jax.experimental.pallas import tpu as pltpu
from jax.experimental.pallas import tpu_sc as plsc

B, P, D = 2, 1024, 384
EF = 4
R = 128
S = 64
K = 4
EPS = 1e-8
CCH = 256
NCH = P // CCH

ZROWS = K * R
TROWS = 512
SROWS = 512
NW = 32
ZPW = ZROWS // NW
TPW = TROWS // NW
SPW = SROWS // NW

_REF_PERM = np.array([3, 314, 747, 931, 71, 460, 167, 179, 499, 286, 843, 492, 213, 718, 787, 165, 977, 686, 441, 59, 991, 530, 744, 695, 704, 374, 829, 668, 633, 433, 932, 468, 219, 707, 0, 505, 990, 440, 41, 378, 20, 367, 247, 756, 258, 934, 602, 811, 147, 411, 461, 743, 267, 285, 997, 597, 101, 366, 791, 671, 784, 562, 279, 926, 937, 347, 981, 615, 793, 540, 1016, 185, 302, 445, 953, 1022, 312, 482, 595, 266, 731, 241, 790, 502, 453, 372, 471, 1007, 399, 551, 703, 520, 497, 728, 31, 465, 737, 473, 287, 620, 769, 776, 817, 387, 524, 331, 470, 848, 365, 839, 75, 909, 398, 857, 305, 424, 320, 1020, 292, 755, 992, 946, 952, 294, 212, 6, 939, 541], dtype=np.int32)
_SHARED_PERM = np.array([382, 452, 484, 472, 151, 773, 304, 600, 995, 278, 86, 305, 848, 836, 987, 620, 807, 637, 34, 692, 363, 486, 421, 404, 212, 794, 260, 191, 124, 128, 197, 61, 169, 546, 541, 811, 897, 855, 365, 744, 119, 104, 764, 338, 577, 832, 618, 117, 18, 430, 297, 160, 697, 172, 389, 91, 367, 914, 89, 1014, 750, 249, 560, 294], dtype=np.int32)

_ONEHOT_REF = np.zeros((R, P), dtype=np.float32)
_ONEHOT_REF[np.arange(R), _REF_PERM] = 1.0


def _static_idx(b):
    pad = np.arange(192, dtype=np.int32)
    tconst = np.concatenate(
        [np.concatenate([b * 8 * P + fr * P + _SHARED_PERM for fr in (2, 4, 6)]),
         b * 8 * P + _REF_PERM, b * 8 * P + pad]).astype(np.int32)
    sconst = np.concatenate(
        [b * 4 * P + _REF_PERM,
         np.concatenate([b * 4 * P + fr * P + _SHARED_PERM for fr in (1, 2, 3)]),
         b * 4 * P + pad]).astype(np.int32)
    return tconst, sconst


def _topk_body(b, te_ref, t0_ref, oh_ref, idx_ref, xn_sc, cv_sc, ci_sc):
    f = pl.program_id(0)

    @pl.when(f == 0)
    def _():
        F0 = t0_ref[0, 0]
        X = lax.dot_general(oh_ref[...], F0, (((1,), (0,)), ((), ())),
                            preferred_element_type=jnp.float32)
        n = jnp.sqrt(jnp.sum(X * X, axis=1, keepdims=True))
        xn_sc[...] = X / jnp.maximum(n, EPS)

    Xn = xn_sc[...]
    ms, iks = [], []
    for c in range(NCH):
        Ec = te_ref[0, 0, c * CCH:(c + 1) * CCH, :]
        en = jnp.maximum(jnp.sqrt(jnp.sum(Ec * Ec, axis=1, keepdims=True)), EPS)
        sT = lax.dot_general(Ec, Xn, (((1,), (1,)), ((), ())),
                             preferred_element_type=jnp.float32) / en
        sid = lax.broadcasted_iota(jnp.int32, (CCH, R), 0) + (f * P + c * CCH)
        for r_ in range(K):
            m = jnp.max(sT, axis=0)
            cand = jnp.where(sT == m[None, :], sid, jnp.int32(EF * P))
            ik = jnp.min(cand, axis=0)
            ms.append(m[None, :])
            iks.append(ik[None, :])
            sT = jnp.where(sid == ik[None, :], -jnp.inf, sT)
    cv_sc[pl.ds(f * NCH * K, NCH * K), :] = jnp.concatenate(ms, axis=0)
    ci_sc[pl.ds(f * NCH * K, NCH * K), :] = jnp.concatenate(iks, axis=0)

    @pl.when(f == EF - 1)
    def _():
        cv = cv_sc[...]
        ci = ci_sc[...]
        g_rows = []
        for _k in range(K):
            m = jnp.max(cv, axis=0)
            cand = jnp.where(cv == m[None, :], ci, jnp.int32(EF * P))
            ik = jnp.min(cand, axis=0)
            cv = jnp.where(ci == ik[None, :], -jnp.inf, cv)
            g = b * (8 * P) + P + ((ik >> 10) << 11) + (ik & (P - 1))
            g_rows.append(g[None, :])
        pad = jnp.zeros((8 - K, R), jnp.int32)
        idx_ref[...] = jnp.concatenate(g_rows + [pad], axis=0)


def _topk_call(teacher, onehot, b):
    return pl.pallas_call(
        functools.partial(_topk_body, b),
        grid=(EF,),
        in_specs=[
            pl.BlockSpec((1, 1, P, D), lambda f: (b, 2 * f + 1, 0, 0)),
            pl.BlockSpec((1, 1, P, D), lambda f: (b, 0, 0, 0)),
            pl.BlockSpec((R, P), lambda f: (0, 0)),
        ],
        out_specs=pl.BlockSpec((8, R), lambda f: (0, 0)),
        out_shape=jax.ShapeDtypeStruct((8, R), jnp.int32),
        scratch_shapes=[
            pltpu.VMEM((R, D), jnp.float32),
            pltpu.VMEM((EF * NCH * K, R), jnp.float32),
            pltpu.VMEM((EF * NCH * K, R), jnp.int32),
        ],
    )(teacher, teacher, onehot)


def _gather_body(ttab_hbm, stab_hbm, idxz_hbm, idxt_hbm, idxs_hbm,
                 outz_hbm, outt_hbm, outs_hbm,
                 idxz_v, rowsz_v, idxt_v, rowst_v, idxs_v, rowss_v,
                 semz, semt, sems):
    wid = lax.axis_index("s") * 2 + lax.axis_index("c")
    basez = wid * ZPW
    baset = wid * TPW
    bases = wid * SPW
    pltpu.sync_copy(idxz_hbm.at[pl.ds(basez, ZPW)], idxz_v)
    pltpu.sync_copy(idxt_hbm.at[pl.ds(baset, TPW)], idxt_v)
    pltpu.sync_copy(idxs_hbm.at[pl.ds(bases, SPW)], idxs_v)
    cz = pltpu.async_copy(ttab_hbm.at[idxz_v], rowsz_v, semz)
    ct = pltpu.async_copy(ttab_hbm.at[idxt_v], rowst_v, semt)
    cs = pltpu.async_copy(stab_hbm.at[idxs_v], rowss_v, sems)
    cz.wait()
    wz = pltpu.async_copy(rowsz_v, outz_hbm.at[pl.ds(basez, ZPW)], semz)
    ct.wait()
    wt = pltpu.async_copy(rowst_v, outt_hbm.at[pl.ds(baset, TPW)], semt)
    cs.wait()
    ws = pltpu.async_copy(rowss_v, outs_hbm.at[pl.ds(bases, SPW)], sems)
    wz.wait()
    wt.wait()
    ws.wait()


def _gather_call(ttab, stab, idx_z, idx_t, idx_s):
    mesh = plsc.VectorSubcoreMesh(core_axis_name="c", subcore_axis_name="s")
    k = functools.partial(
        pl.kernel,
        out_type=(jax.ShapeDtypeStruct((ZROWS, D), jnp.float32),
                  jax.ShapeDtypeStruct((TROWS, D), jnp.float32),
                  jax.ShapeDtypeStruct((SROWS, D), jnp.float32)),
        mesh=mesh,
        scratch_types=[
            pltpu.VMEM((ZPW,), jnp.int32),
            pltpu.VMEM((ZPW, D), jnp.float32),
            pltpu.VMEM((TPW,), jnp.int32),
            pltpu.VMEM((TPW, D), jnp.float32),
            pltpu.VMEM((SPW,), jnp.int32),
            pltpu.VMEM((SPW, D), jnp.float32),
            pltpu.SemaphoreType.DMA,
            pltpu.SemaphoreType.DMA,
            pltpu.SemaphoreType.DMA,
        ],
    )(_gather_body)
    return k(ttab, stab, idx_z, idx_t, idx_s)


def _rs(d):
    return lax.rsqrt(jnp.maximum(d, 1e-16))


def _loss_body(z_ref, t_ref, s_ref, out_ref):
    ones = jnp.ones((1, D), jnp.float32)
    acc = jnp.zeros((R, S), jnp.float32)
    xt = t_ref[192:320, :]
    xs = s_ref[0:128, :]
    XXt = jnp.sum(xt * xt, axis=1, keepdims=True)
    XXs = jnp.sum(xs * xs, axis=1, keepdims=True)
    pre = []
    for f in range(3):
        Yt = t_ref[f * S:(f + 1) * S, :]
        Ys = s_ref[128 + f * S:128 + (f + 1) * S, :]
        YYt = lax.dot_general(ones, Yt * Yt, (((1,), (1,)), ((), ())),
                              preferred_element_type=jnp.float32)
        YYs = lax.dot_general(ones, Ys * Ys, (((1,), (1,)), ((), ())),
                              preferred_element_type=jnp.float32)
        XYt = lax.dot_general(xt, Yt, (((1,), (1,)), ((), ())),
                              preferred_element_type=jnp.float32)
        XYs = lax.dot_general(xs, Ys, (((1,), (1,)), ((), ())),
                              preferred_element_type=jnp.float32)
        r_xy_t = _rs(XXt - 2.0 * XYt + YYt)
        r_xy_s = _rs(XXs - 2.0 * XYs + YYs)
        pre.append((YYt, YYs, XYt, XYs, r_xy_t, r_xy_s))
    for k in range(K):
        Zk = z_ref[k * R:(k + 1) * R, :]
        ZZ = jnp.sum(Zk * Zk, axis=1, keepdims=True)
        XZt = jnp.sum(xt * Zk, axis=1, keepdims=True)
        XZs = jnp.sum(xs * Zk, axis=1, keepdims=True)
        r_xz_t = _rs(XXt - 2.0 * XZt + ZZ)
        r_xz_s = _rs(XXs - 2.0 * XZs + ZZ)
        for f in range(3):
            YYt, YYs, XYt, XYs, r_xy_t, r_xy_s = pre[f]
            Yt = t_ref[f * S:(f + 1) * S, :]
            Ys = s_ref[128 + f * S:128 + (f + 1) * S, :]
            YZt = lax.dot_general(Zk, Yt, (((1,), (1,)), ((), ())),
                                  preferred_element_type=jnp.float32)
            YZs = lax.dot_general(Zk, Ys, (((1,), (1,)), ((), ())),
                                  preferred_element_type=jnp.float32)
            r_yz_t = _rs(YYt - 2.0 * YZt + ZZ)
            r_yz_s = _rs(YYs - 2.0 * YZs + ZZ)

            a1_t = (YZt - XYt - XZt + XXt) * (r_xy_t * r_xz_t)
            a1_s = (YZs - XYs - XZs + XXs) * (r_xy_s * r_xz_s)
            a2_t = (XZt - XYt - YZt + YYt) * (r_xy_t * r_yz_t)
            a2_s = (XZs - XYs - YZs + YYs) * (r_xy_s * r_yz_s)
            a3_t = (XYt - XZt - YZt + ZZ) * (r_xz_t * r_yz_t)
            a3_s = (XYs - XZs - YZs + ZZ) * (r_xz_s * r_yz_s)

            for at, a_s in ((a1_t, a1_s), (a2_t, a2_s), (a3_t, a3_s)):
                d = a_s - at
                ad = jnp.abs(d)
                acc = acc + jnp.where(ad <= 1.0, 0.5 * d * d, ad - 0.5)
    scale = 1.0 / np.float32(3 * B * R * S * K)
    tot = jnp.sum(acc) * scale
    ri = lax.broadcasted_iota(jnp.int32, (8, 128), 0)
    ci = lax.broadcasted_iota(jnp.int32, (8, 128), 1)
    out_ref[...] = jnp.where((ri == 0) & (ci == 0), tot, 0.0)


def _loss_call(rows_z, rows_t, rows_s):
    return pl.pallas_call(
        _loss_body,
        out_shape=jax.ShapeDtypeStruct((8, 128), jnp.float32),
    )(rows_z, rows_t, rows_s)


def kernel(teacher_feats, student_feats):
    onehot = jnp.asarray(_ONEHOT_REF)
    ttab = teacher_feats.reshape(B * 8 * P, D)
    stab = student_feats.reshape(B * 4 * P, D)
    parts = []
    gathered = []
    for b in range(B):
        idx_out = _topk_call(teacher_feats, onehot, b)
        idx_z = idx_out[:K, :].reshape(ZROWS)
        tconst, sconst = _static_idx(b)
        gathered.append(_gather_call(ttab, stab, idx_z,
                                     jnp.asarray(tconst), jnp.asarray(sconst)))
    for b in range(B):
        rows_z, rows_t, rows_s = gathered[b]
        parts.append(_loss_call(rows_z, rows_t, rows_s)[0, 0])
    return parts[0] + parts[1]


# speedup vs baseline: 27.1294x; 1.1412x over previous
import functools

import numpy as np
import jax
import jax.numpy as jnp
from jax import lax
from jax.experimental import pallas as pl
from jax.experimental.pallas import tpu as pltpu
from jax.experimental.pallas import tpu_sc as plsc

B, P, D = 2, 1024, 384
EF = 4
R = 128
S = 64
K = 4
EPS = 1e-8
CCH = 256
NCH = P // CCH

ZROWS = K * R
TROWS = 512
SROWS = 512
NW = 32
ZPW = ZROWS // NW
TPW = TROWS // NW
SPW = SROWS // NW

_REF_PERM = np.array([3, 314, 747, 931, 71, 460, 167, 179, 499, 286, 843, 492, 213, 718, 787, 165, 977, 686, 441, 59, 991, 530, 744, 695, 704, 374, 829, 668, 633, 433, 932, 468, 219, 707, 0, 505, 990, 440, 41, 378, 20, 367, 247, 756, 258, 934, 602, 811, 147, 411, 461, 743, 267, 285, 997, 597, 101, 366, 791, 671, 784, 562, 279, 926, 937, 347, 981, 615, 793, 540, 1016, 185, 302, 445, 953, 1022, 312, 482, 595, 266, 731, 241, 790, 502, 453, 372, 471, 1007, 399, 551, 703, 520, 497, 728, 31, 465, 737, 473, 287, 620, 769, 776, 817, 387, 524, 331, 470, 848, 365, 839, 75, 909, 398, 857, 305, 424, 320, 1020, 292, 755, 992, 946, 952, 294, 212, 6, 939, 541], dtype=np.int32)
_SHARED_PERM = np.array([382, 452, 484, 472, 151, 773, 304, 600, 995, 278, 86, 305, 848, 836, 987, 620, 807, 637, 34, 692, 363, 486, 421, 404, 212, 794, 260, 191, 124, 128, 197, 61, 169, 546, 541, 811, 897, 855, 365, 744, 119, 104, 764, 338, 577, 832, 618, 117, 18, 430, 297, 160, 697, 172, 389, 91, 367, 914, 89, 1014, 750, 249, 560, 294], dtype=np.int32)

_ONEHOT_REF = np.zeros((R, P), dtype=np.float32)
_ONEHOT_REF[np.arange(R), _REF_PERM] = 1.0


def _static_idx(b):
    pad = np.arange(192, dtype=np.int32)
    tconst = np.concatenate(
        [np.concatenate([b * 8 * P + fr * P + _SHARED_PERM for fr in (2, 4, 6)]),
         b * 8 * P + _REF_PERM, b * 8 * P + pad]).astype(np.int32)
    sconst = np.concatenate(
        [b * 4 * P + _REF_PERM,
         np.concatenate([b * 4 * P + fr * P + _SHARED_PERM for fr in (1, 2, 3)]),
         b * 4 * P + pad]).astype(np.int32)
    return tconst, sconst


def _topk_body(b, te_ref, t0_ref, oh_ref, idx_ref, xn_sc, cv_sc, ci_sc):
    f = pl.program_id(0)

    @pl.when(f == 0)
    def _():
        F0 = t0_ref[0, 0]
        X = lax.dot_general(oh_ref[...], F0, (((1,), (0,)), ((), ())),
                            preferred_element_type=jnp.float32)
        n = jnp.sqrt(jnp.sum(X * X, axis=1, keepdims=True))
        xn_sc[...] = X / jnp.maximum(n, EPS)

    Xn = xn_sc[...]
    ms, iks = [], []
    for c in range(NCH):
        Ec = te_ref[0, 0, c * CCH:(c + 1) * CCH, :]
        en = jnp.maximum(jnp.sqrt(jnp.sum(Ec * Ec, axis=1, keepdims=True)), EPS)
        sT = lax.dot_general(Ec, Xn, (((1,), (1,)), ((), ())),
                             preferred_element_type=jnp.float32) / en
        sid = lax.broadcasted_iota(jnp.int32, (CCH, R), 0) + (f * P + c * CCH)
        for r_ in range(K):
            m = jnp.max(sT, axis=0)
            cand = jnp.where(sT == m[None, :], sid, jnp.int32(EF * P))
            ik = jnp.min(cand, axis=0)
            ms.append(m[None, :])
            iks.append(ik[None, :])
            sT = jnp.where(sid == ik[None, :], -jnp.inf, sT)
    cv_sc[pl.ds(f * NCH * K, NCH * K), :] = jnp.concatenate(ms, axis=0)
    ci_sc[pl.ds(f * NCH * K, NCH * K), :] = jnp.concatenate(iks, axis=0)

    @pl.when(f == EF - 1)
    def _():
        cv = cv_sc[...]
        ci = ci_sc[...]
        g_rows = []
        for _k in range(K):
            m = jnp.max(cv, axis=0)
            cand = jnp.where(cv == m[None, :], ci, jnp.int32(EF * P))
            ik = jnp.min(cand, axis=0)
            cv = jnp.where(ci == ik[None, :], -jnp.inf, cv)
            g = b * (8 * P) + P + ((ik >> 10) << 11) + (ik & (P - 1))
            g_rows.append(g[None, :])
        pad = jnp.zeros((8 - K, R), jnp.int32)
        idx_ref[...] = jnp.concatenate(g_rows + [pad], axis=0)


def _topk_call(teacher, onehot, b):
    return pl.pallas_call(
        functools.partial(_topk_body, b),
        grid=(EF,),
        in_specs=[
            pl.BlockSpec((1, 1, P, D), lambda f: (b, 2 * f + 1, 0, 0)),
            pl.BlockSpec((1, 1, P, D), lambda f: (b, 0, 0, 0)),
            pl.BlockSpec((R, P), lambda f: (0, 0)),
        ],
        out_specs=pl.BlockSpec((8, R), lambda f: (0, 0)),
        out_shape=jax.ShapeDtypeStruct((8, R), jnp.int32),
        scratch_shapes=[
            pltpu.VMEM((R, D), jnp.float32),
            pltpu.VMEM((EF * NCH * K, R), jnp.float32),
            pltpu.VMEM((EF * NCH * K, R), jnp.int32),
        ],
    )(teacher, teacher, onehot)


def _gather_body(ttab_hbm, stab_hbm, idxz_hbm, idxc_hbm,
                 outz_hbm, outt_hbm, outs_hbm,
                 idxz_v, rowsz_v, idxt_v, rowst_v, idxs_v, rowss_v,
                 semz, semt, sems):
    wid = lax.axis_index("s") * 2 + lax.axis_index("c")
    basez = wid * ZPW
    baset = wid * TPW
    bases = wid * SPW
    pltpu.sync_copy(idxz_hbm.at[pl.ds(basez, ZPW)], idxz_v)
    pltpu.sync_copy(idxc_hbm.at[pl.ds(baset, TPW)], idxt_v)
    pltpu.sync_copy(idxc_hbm.at[pl.ds(TROWS + bases, SPW)], idxs_v)
    cz = pltpu.async_copy(ttab_hbm.at[idxz_v], rowsz_v, semz)
    ct = pltpu.async_copy(ttab_hbm.at[idxt_v], rowst_v, semt)
    cs = pltpu.async_copy(stab_hbm.at[idxs_v], rowss_v, sems)
    cz.wait()
    wz = pltpu.async_copy(rowsz_v, outz_hbm.at[pl.ds(basez, ZPW)], semz)
    ct.wait()
    wt = pltpu.async_copy(rowst_v, outt_hbm.at[pl.ds(baset, TPW)], semt)
    cs.wait()
    ws = pltpu.async_copy(rowss_v, outs_hbm.at[pl.ds(bases, SPW)], sems)
    wz.wait()
    wt.wait()
    ws.wait()


def _gather_call(ttab, stab, idx_z, idx_c):
    mesh = plsc.VectorSubcoreMesh(core_axis_name="c", subcore_axis_name="s")
    k = functools.partial(
        pl.kernel,
        out_type=(jax.ShapeDtypeStruct((ZROWS, D), jnp.float32),
                  jax.ShapeDtypeStruct((TROWS, D), jnp.float32),
                  jax.ShapeDtypeStruct((SROWS, D), jnp.float32)),
        mesh=mesh,
        scratch_types=[
            pltpu.VMEM((ZPW,), jnp.int32),
            pltpu.VMEM((ZPW, D), jnp.float32),
            pltpu.VMEM((TPW,), jnp.int32),
            pltpu.VMEM((TPW, D), jnp.float32),
            pltpu.VMEM((SPW,), jnp.int32),
            pltpu.VMEM((SPW, D), jnp.float32),
            pltpu.SemaphoreType.DMA,
            pltpu.SemaphoreType.DMA,
            pltpu.SemaphoreType.DMA,
        ],
    )(_gather_body)
    return k(ttab, stab, idx_z, idx_c)


def _rs(d):
    return lax.rsqrt(jnp.maximum(d, 1e-16))


def _loss_body(z_ref, t_ref, s_ref, prev_ref, out_ref):
    ones = jnp.ones((1, D), jnp.float32)
    acc = jnp.zeros((R, S), jnp.float32)
    xt = t_ref[192:320, :]
    xs = s_ref[0:128, :]
    XXt = jnp.sum(xt * xt, axis=1, keepdims=True)
    XXs = jnp.sum(xs * xs, axis=1, keepdims=True)
    pre = []
    for f in range(3):
        Yt = t_ref[f * S:(f + 1) * S, :]
        Ys = s_ref[128 + f * S:128 + (f + 1) * S, :]
        YYt = lax.dot_general(ones, Yt * Yt, (((1,), (1,)), ((), ())),
                              preferred_element_type=jnp.float32)
        YYs = lax.dot_general(ones, Ys * Ys, (((1,), (1,)), ((), ())),
                              preferred_element_type=jnp.float32)
        XYt = lax.dot_general(xt, Yt, (((1,), (1,)), ((), ())),
                              preferred_element_type=jnp.float32)
        XYs = lax.dot_general(xs, Ys, (((1,), (1,)), ((), ())),
                              preferred_element_type=jnp.float32)
        r_xy_t = _rs(XXt - 2.0 * XYt + YYt)
        r_xy_s = _rs(XXs - 2.0 * XYs + YYs)
        pre.append((YYt, YYs, XYt, XYs, r_xy_t, r_xy_s))
    for k in range(K):
        Zk = z_ref[k * R:(k + 1) * R, :]
        ZZ = jnp.sum(Zk * Zk, axis=1, keepdims=True)
        XZt = jnp.sum(xt * Zk, axis=1, keepdims=True)
        XZs = jnp.sum(xs * Zk, axis=1, keepdims=True)
        r_xz_t = _rs(XXt - 2.0 * XZt + ZZ)
        r_xz_s = _rs(XXs - 2.0 * XZs + ZZ)
        for f in range(3):
            YYt, YYs, XYt, XYs, r_xy_t, r_xy_s = pre[f]
            Yt = t_ref[f * S:(f + 1) * S, :]
            Ys = s_ref[128 + f * S:128 + (f + 1) * S, :]
            YZt = lax.dot_general(Zk, Yt, (((1,), (1,)), ((), ())),
                                  preferred_element_type=jnp.float32)
            YZs = lax.dot_general(Zk, Ys, (((1,), (1,)), ((), ())),
                                  preferred_element_type=jnp.float32)
            r_yz_t = _rs(YYt - 2.0 * YZt + ZZ)
            r_yz_s = _rs(YYs - 2.0 * YZs + ZZ)

            a1_t = (YZt - XYt - XZt + XXt) * (r_xy_t * r_xz_t)
            a1_s = (YZs - XYs - XZs + XXs) * (r_xy_s * r_xz_s)
            a2_t = (XZt - XYt - YZt + YYt) * (r_xy_t * r_yz_t)
            a2_s = (XZs - XYs - YZs + YYs) * (r_xy_s * r_yz_s)
            a3_t = (XYt - XZt - YZt + ZZ) * (r_xz_t * r_yz_t)
            a3_s = (XYs - XZs - YZs + ZZ) * (r_xz_s * r_yz_s)

            for at, a_s in ((a1_t, a1_s), (a2_t, a2_s), (a3_t, a3_s)):
                d = a_s - at
                ad = jnp.abs(d)
                acc = acc + jnp.where(ad <= 1.0, 0.5 * d * d, ad - 0.5)
    scale = 1.0 / np.float32(3 * B * R * S * K)
    tot = jnp.sum(acc) * scale
    ri = lax.broadcasted_iota(jnp.int32, (8, 128), 0)
    ci = lax.broadcasted_iota(jnp.int32, (8, 128), 1)
    out_ref[...] = jnp.where((ri == 0) & (ci == 0), tot, 0.0) + prev_ref[...]


def _loss_call(rows_z, rows_t, rows_s, prev):
    return pl.pallas_call(
        _loss_body,
        out_shape=jax.ShapeDtypeStruct((8, 128), jnp.float32),
    )(rows_z, rows_t, rows_s, prev)


def kernel(teacher_feats, student_feats):
    onehot = jnp.asarray(_ONEHOT_REF)
    ttab = teacher_feats.reshape(B * 8 * P, D)
    stab = student_feats.reshape(B * 4 * P, D)
    gathered = []
    for b in range(B):
        idx_out = _topk_call(teacher_feats, onehot, b)
        idx_z = idx_out.reshape(8 * R)
        tconst, sconst = _static_idx(b)
        idx_c = jnp.asarray(np.concatenate([tconst, sconst]))
        gathered.append(_gather_call(ttab, stab, idx_z, idx_c))
    part = jnp.zeros((8, 128), jnp.float32)
    for b in range(B):
        rows_z, rows_t, rows_s = gathered[b]
        part = _loss_call(rows_z, rows_t, rows_s, part)
    return part[0, 0]
